# baseline (speedup 1.0000x reference)
import dataclasses

import jax
import jax.numpy as jnp
from jax import lax
from jax.experimental import pallas as pl
from jax.experimental.pallas import tpu as pltpu
from jax.experimental.pallas import tpu_sc as plsc

N_NODES = 10000
N_EDGES = 320000
DIM = 128
NUM_RELS = 5

NPAD = 10240
NUM_TILES = 32
EDGES_PER_TILE = N_EDGES // NUM_TILES
CHUNK = 128
EDGES_PER_TILE_PAD = 10240
CHUNKS_PER_TILE = EDGES_PER_TILE_PAD // CHUNK
SUPER = 4
NSUP = CHUNKS_PER_TILE // SUPER
RING = 2
ROWS_PER_TILE = NPAD // 16


_HP_BLK = 1000


def _hp_body(h_ref, rv_ref, out_ref):
    hb = h_ref[...]
    for r in range(NUM_RELS):
        out_ref[r] = jnp.maximum(hb + rv_ref[r], 0.0)


def _build_hp(h, relvectors):
    return pl.pallas_call(
        _hp_body,
        grid=(N_NODES // _HP_BLK,),
        in_specs=[
            pl.BlockSpec((_HP_BLK, DIM), lambda i: (i, 0)),
            pl.BlockSpec((NUM_RELS, DIM), lambda i: (0, 0)),
        ],
        out_specs=pl.BlockSpec((NUM_RELS, _HP_BLK, DIM), lambda i: (0, i, 0)),
        out_shape=jax.ShapeDtypeStruct((NUM_RELS, N_NODES, DIM), jnp.float32),
    )(h, relvectors)


def _sc_body(hp_hbm, src_hbm, dst_hbm, eid_hbm, z_hbm,
             acc0_hbm, acc1_hbm, cnt_hbm,
             gA, gB, dA, dB, eA, eB, rows0, rows1,
             cnt_v, acc_sh, gs0, gs1, isA, isB):
    gbufs = (gA, gB)
    dbufs = (dA, dB)
    ebufs = (eA, eB)
    rows = (rows0, rows1)
    gsems = (gs0, gs1)
    isems = (isA, isB)
    c = lax.axis_index("c")
    s = lax.axis_index("s")
    wid = c * 16 + s
    row0 = s * ROWS_PER_TILE

    pltpu.sync_copy(z_hbm, acc_sh.at[pl.ds(row0, ROWS_PER_TILE)])

    zeros16 = jnp.zeros((16,), jnp.float32)
    zidx16 = jnp.zeros((16,), jnp.int32)
    ones16 = jnp.ones((16,), jnp.float32)

    @pl.loop(0, NPAD, step=16)
    def _(i):
        cnt_v[0, pl.ds(i, 16)] = zeros16

    def load_idx(sup, p):
        sl = pl.ds(sup * SUPER, SUPER)
        pltpu.async_copy(src_hbm.at[wid, sl], gbufs[p], isems[p])
        pltpu.async_copy(dst_hbm.at[wid, sl], dbufs[p], isems[p])
        pltpu.async_copy(eid_hbm.at[wid, sl], ebufs[p], isems[p])

    def wait_idx(p):
        for _ in range(3):
            pltpu.make_async_copy(src_hbm.at[wid, pl.ds(0, SUPER)],
                                  gbufs[p], isems[p]).wait()

    def compute_idx(p):
        @pl.loop(0, SUPER)
        def _(k):
            @pl.loop(0, CHUNK, step=16)
            def _(j):
                sv = gbufs[p][k, pl.ds(j, 16)]
                ev = ebufs[p][k, pl.ds(j, 16)]
                gbufs[p][k, pl.ds(j, 16)] = ev * N_NODES + sv
                plsc.addupdate_scatter(
                    cnt_v, [zidx16, dbufs[p][k, pl.ds(j, 16)]], ones16)

    load_idx(0, 0)
    wait_idx(0)
    compute_idx(0)
    plsc.subcore_barrier()
    for b in range(RING):
        pltpu.async_copy(hp_hbm.at[gA.at[b]], rows[b], gsems[b])
    load_idx(1, 1)

    def super_body(sup, p):
        q = 1 - p
        for k in range(SUPER):
            b = k % RING
            pltpu.make_async_copy(hp_hbm.at[pl.ds(0, CHUNK)], rows[b],
                                  gsems[b]).wait()
            pltpu.sync_copy(rows[b], acc_sh.at[dbufs[p].at[k]], add=True)
            if k < SUPER - RING:
                pltpu.async_copy(hp_hbm.at[gbufs[p].at[k + RING]], rows[b],
                                 gsems[b])
            else:
                @pl.when(sup + 1 < NSUP)
                def _():
                    pltpu.async_copy(
                        hp_hbm.at[gbufs[q].at[k + RING - SUPER]], rows[b],
                        gsems[b])
            if k == 1:
                @pl.when(sup + 1 < NSUP)
                def _():
                    wait_idx(q)
                    compute_idx(q)
        @pl.when(sup + 2 < NSUP)
        def _():
            load_idx(sup + 2, p)

    @pl.loop(0, NSUP, step=2)
    def _(sup):
        super_body(sup, 0)
        super_body(sup + 1, 1)

    plsc.subcore_barrier()

    pltpu.sync_copy(cnt_v, cnt_hbm.at[wid])

    @pl.when(c == 0)
    def _():
        pltpu.sync_copy(acc_sh.at[pl.ds(row0, ROWS_PER_TILE)],
                        acc0_hbm.at[pl.ds(row0, ROWS_PER_TILE)])

    @pl.when(c == 1)
    def _():
        pltpu.sync_copy(acc_sh.at[pl.ds(row0, ROWS_PER_TILE)],
                        acc1_hbm.at[pl.ds(row0, ROWS_PER_TILE)])


def _sc_aggregate(hp, src, dst, eid, zrows):
    mesh = plsc.VectorSubcoreMesh(core_axis_name="c", subcore_axis_name="s")
    out_type = (
        jax.ShapeDtypeStruct((NPAD, DIM), jnp.float32),
        jax.ShapeDtypeStruct((NPAD, DIM), jnp.float32),
        jax.ShapeDtypeStruct((NUM_TILES, 1, NPAD), jnp.float32),
    )
    scratch = (
        [pltpu.VMEM((SUPER, CHUNK), jnp.int32) for _ in range(6)]
        + [pltpu.VMEM((CHUNK, DIM), jnp.float32) for _ in range(2)]
        + [
            pltpu.VMEM((1, NPAD), jnp.float32),
            pltpu.VMEM_SHARED((NPAD, DIM), jnp.float32),
        ]
        + [pltpu.SemaphoreType.DMA for _ in range(4)]
    )
    cp = pltpu.CompilerParams()
    if "needs_layout_passes" in pltpu.CompilerParams.__dataclass_fields__:
        cp = dataclasses.replace(cp, needs_layout_passes=False)
    fn = pl.kernel(_sc_body, out_type=out_type, mesh=mesh,
                   scratch_types=scratch, compiler_params=cp)
    return fn(hp, src, dst, eid, zrows)


_FIN_BLK = 1024


def _fin_body(a0_ref, a1_ref, cnt_ref, w_ref, b_ref, out_ref):
    ssum = a0_ref[...] + a1_ref[...]
    csum = jnp.sum(cnt_ref[...], axis=0)
    eye = (lax.broadcasted_iota(jnp.int32, (DIM, DIM), 0)
           == lax.broadcasted_iota(jnp.int32, (DIM, DIM), 1)
           ).astype(jnp.float32)
    ct = lax.dot_general(eye, csum, (((1,), (1,)), ((), ())),
                         preferred_element_type=jnp.float32)
    pieces = []
    for k in range(_FIN_BLK // DIM):
        col = jnp.maximum(ct[:, k:k + 1], 1.0)
        pieces.append(ssum[k * DIM:(k + 1) * DIM, :] / col)
    red = jnp.concatenate(pieces, axis=0)
    out_ref[...] = lax.dot_general(
        red, w_ref[...], (((1,), (1,)), ((), ())),
        preferred_element_type=jnp.float32) + b_ref[...]


def _finalize(acc0, acc1, cnts, W, b2):
    return pl.pallas_call(
        _fin_body,
        grid=(NPAD // _FIN_BLK,),
        in_specs=[
            pl.BlockSpec((_FIN_BLK, DIM), lambda g: (g, 0)),
            pl.BlockSpec((_FIN_BLK, DIM), lambda g: (g, 0)),
            pl.BlockSpec((NUM_TILES, _FIN_BLK // DIM, DIM), lambda g: (0, g, 0)),
            pl.BlockSpec((DIM, DIM), lambda g: (0, 0)),
            pl.BlockSpec((1, DIM), lambda g: (0, 0)),
        ],
        out_specs=pl.BlockSpec((_FIN_BLK, DIM), lambda g: (g, 0)),
        out_shape=jax.ShapeDtypeStruct((NPAD, DIM), jnp.float32),
    )(acc0, acc1, cnts, W, b2)


@jax.jit
def kernel(h, edge_index, edge_id, W, b, relvectors):
    src = edge_index[0].astype(jnp.int32)
    dst = edge_index[1].astype(jnp.int32)
    eid = edge_id.astype(jnp.int32)
    hp = _build_hp(h, relvectors).reshape(NUM_RELS * N_NODES, DIM)
    npad_e = EDGES_PER_TILE_PAD - EDGES_PER_TILE
    src = jnp.pad(src.reshape(NUM_TILES, EDGES_PER_TILE), ((0, 0), (0, npad_e))
                  ).reshape(NUM_TILES, CHUNKS_PER_TILE, CHUNK)
    eid = jnp.pad(eid.reshape(NUM_TILES, EDGES_PER_TILE), ((0, 0), (0, npad_e))
                  ).reshape(NUM_TILES, CHUNKS_PER_TILE, CHUNK)
    dst = jnp.pad(dst.reshape(NUM_TILES, EDGES_PER_TILE), ((0, 0), (0, npad_e)),
                  constant_values=N_NODES
                  ).reshape(NUM_TILES, CHUNKS_PER_TILE, CHUNK)
    zrows = jnp.zeros((ROWS_PER_TILE, DIM), jnp.float32)
    acc0, acc1, cnts = _sc_aggregate(hp, src, dst, eid, zrows)
    cnts = cnts.reshape(NUM_TILES, NPAD // DIM, DIM)
    out = _finalize(acc0, acc1, cnts, W, b.reshape(1, DIM))
    return out[:N_NODES]


# speedup vs baseline: 1.0743x; 1.0743x over previous
import dataclasses

import jax
import jax.numpy as jnp
from jax import lax
from jax.experimental import pallas as pl
from jax.experimental.pallas import tpu as pltpu
from jax.experimental.pallas import tpu_sc as plsc

N_NODES = 10000
N_EDGES = 320000
DIM = 128
NUM_RELS = 5

NPAD = 10240
NUM_TILES = 32
EDGES_PER_TILE = N_EDGES // NUM_TILES
CHUNK = 80
CHUNKS_PER_TILE = EDGES_PER_TILE // CHUNK
ROWS_PER_TILE = NPAD // 16


_HP_BLK = 1000


def _hp_body(h_ref, rv_ref, out_ref):
    hb = h_ref[...]
    for r in range(NUM_RELS):
        out_ref[r] = jnp.maximum(hb + rv_ref[r], 0.0)


def _build_hp(h, relvectors):
    return pl.pallas_call(
        _hp_body,
        grid=(N_NODES // _HP_BLK,),
        in_specs=[
            pl.BlockSpec((_HP_BLK, DIM), lambda i: (i, 0)),
            pl.BlockSpec((NUM_RELS, DIM), lambda i: (0, 0)),
        ],
        out_specs=pl.BlockSpec((NUM_RELS, _HP_BLK, DIM), lambda i: (0, i, 0)),
        out_shape=jax.ShapeDtypeStruct((NUM_RELS, N_NODES, DIM), jnp.float32),
    )(h, relvectors)


def _sc_body(hp_hbm, src_hbm, dst_hbm, eid_hbm, z_hbm,
             acc0_hbm, acc1_hbm, cnt_hbm,
             src_v, dst_v, eid_v, gidx_v, rows_v, cnt_v, acc_sh, sem):
    c = lax.axis_index("c")
    s = lax.axis_index("s")
    wid = c * 16 + s
    row0 = s * ROWS_PER_TILE

    pltpu.sync_copy(z_hbm, acc_sh.at[pl.ds(row0, ROWS_PER_TILE)])

    zeros16 = jnp.zeros((16,), jnp.float32)
    zidx16 = jnp.zeros((16,), jnp.int32)
    ones16 = jnp.ones((16,), jnp.float32)

    @pl.loop(0, NPAD, step=16)
    def _(i):
        cnt_v[0, pl.ds(i, 16)] = zeros16

    plsc.subcore_barrier()

    base_w = wid * EDGES_PER_TILE

    @pl.loop(0, CHUNKS_PER_TILE)
    def _(ci):
        base = base_w + ci * CHUNK
        pltpu.sync_copy(src_hbm.at[pl.ds(base, CHUNK)], src_v)
        pltpu.sync_copy(dst_hbm.at[pl.ds(base, CHUNK)], dst_v)
        pltpu.sync_copy(eid_hbm.at[pl.ds(base, CHUNK)], eid_v)

        @pl.loop(0, CHUNK, step=16)
        def _(j):
            sv = src_v[pl.ds(j, 16)]
            ev = eid_v[pl.ds(j, 16)]
            gidx_v[pl.ds(j, 16)] = ev * N_NODES + sv
            plsc.addupdate_scatter(cnt_v, [zidx16, dst_v[pl.ds(j, 16)]],
                                   ones16)

        pltpu.async_copy(hp_hbm.at[gidx_v], rows_v, sem).wait()
        pltpu.sync_copy(rows_v, acc_sh.at[dst_v], add=True)

    plsc.subcore_barrier()

    pltpu.sync_copy(cnt_v, cnt_hbm.at[wid])

    @pl.when(c == 0)
    def _():
        pltpu.sync_copy(acc_sh.at[pl.ds(row0, ROWS_PER_TILE)],
                        acc0_hbm.at[pl.ds(row0, ROWS_PER_TILE)])

    @pl.when(c == 1)
    def _():
        pltpu.sync_copy(acc_sh.at[pl.ds(row0, ROWS_PER_TILE)],
                        acc1_hbm.at[pl.ds(row0, ROWS_PER_TILE)])


def _sc_aggregate(hp, src, dst, eid, zrows):
    mesh = plsc.VectorSubcoreMesh(core_axis_name="c", subcore_axis_name="s")
    out_type = (
        jax.ShapeDtypeStruct((NPAD, DIM), jnp.float32),
        jax.ShapeDtypeStruct((NPAD, DIM), jnp.float32),
        jax.ShapeDtypeStruct((NUM_TILES, 1, NPAD), jnp.float32),
    )
    scratch = [
        pltpu.VMEM((CHUNK,), jnp.int32),
        pltpu.VMEM((CHUNK,), jnp.int32),
        pltpu.VMEM((CHUNK,), jnp.int32),
        pltpu.VMEM((CHUNK,), jnp.int32),
        pltpu.VMEM((CHUNK, DIM), jnp.float32),
        pltpu.VMEM((1, NPAD), jnp.float32),
        pltpu.VMEM_SHARED((NPAD, DIM), jnp.float32),
        pltpu.SemaphoreType.DMA,
    ]
    cp = pltpu.CompilerParams()
    if "needs_layout_passes" in pltpu.CompilerParams.__dataclass_fields__:
        cp = dataclasses.replace(cp, needs_layout_passes=False)
    fn = pl.kernel(_sc_body, out_type=out_type, mesh=mesh,
                   scratch_types=scratch, compiler_params=cp)
    return fn(hp, src, dst, eid, zrows)


_FIN_BLK = 1024


def _fin_body(a0_ref, a1_ref, cnt_ref, w_ref, b_ref, out_ref):
    ssum = a0_ref[...] + a1_ref[...]
    csum = jnp.sum(cnt_ref[...], axis=0)
    eye = (lax.broadcasted_iota(jnp.int32, (DIM, DIM), 0)
           == lax.broadcasted_iota(jnp.int32, (DIM, DIM), 1)
           ).astype(jnp.float32)
    ct = lax.dot_general(eye, csum, (((1,), (1,)), ((), ())),
                         preferred_element_type=jnp.float32)
    pieces = []
    for k in range(_FIN_BLK // DIM):
        col = jnp.maximum(ct[:, k:k + 1], 1.0)
        pieces.append(ssum[k * DIM:(k + 1) * DIM, :] / col)
    red = jnp.concatenate(pieces, axis=0)
    out_ref[...] = lax.dot_general(
        red, w_ref[...], (((1,), (1,)), ((), ())),
        preferred_element_type=jnp.float32) + b_ref[...]


def _finalize(acc0, acc1, cnts, W, b2):
    return pl.pallas_call(
        _fin_body,
        grid=(NPAD // _FIN_BLK,),
        in_specs=[
            pl.BlockSpec((_FIN_BLK, DIM), lambda g: (g, 0)),
            pl.BlockSpec((_FIN_BLK, DIM), lambda g: (g, 0)),
            pl.BlockSpec((NUM_TILES, _FIN_BLK // DIM, DIM), lambda g: (0, g, 0)),
            pl.BlockSpec((DIM, DIM), lambda g: (0, 0)),
            pl.BlockSpec((1, DIM), lambda g: (0, 0)),
        ],
        out_specs=pl.BlockSpec((_FIN_BLK, DIM), lambda g: (g, 0)),
        out_shape=jax.ShapeDtypeStruct((NPAD, DIM), jnp.float32),
    )(acc0, acc1, cnts, W, b2)


@jax.jit
def kernel(h, edge_index, edge_id, W, b, relvectors):
    src = edge_index[0].astype(jnp.int32)
    dst = edge_index[1].astype(jnp.int32)
    eid = edge_id.astype(jnp.int32)
    hp = _build_hp(h, relvectors).reshape(NUM_RELS * N_NODES, DIM)
    zrows = jnp.zeros((ROWS_PER_TILE, DIM), jnp.float32)
    acc0, acc1, cnts = _sc_aggregate(hp, src, dst, eid, zrows)
    cnts = cnts.reshape(NUM_TILES, NPAD // DIM, DIM)
    out = _finalize(acc0, acc1, cnts, W, b.reshape(1, DIM))
    return out[:N_NODES]


# speedup vs baseline: 1.3086x; 1.2181x over previous
import dataclasses

import jax
import jax.numpy as jnp
from jax import lax
from jax.experimental import pallas as pl
from jax.experimental.pallas import tpu as pltpu
from jax.experimental.pallas import tpu_sc as plsc

N_NODES = 10000
N_EDGES = 320000
DIM = 128
NUM_RELS = 5

NPAD = 10240
NUM_TILES = 32
EDGES_PER_TILE = N_EDGES // NUM_TILES
CHUNK = 80
CHUNKS_PER_TILE = EDGES_PER_TILE // CHUNK
ROWS_PER_TILE = NPAD // 16


_HP_BLK = 1000


def _hp_body(h_ref, rv_ref, out_ref):
    hb = h_ref[...]
    for r in range(NUM_RELS):
        out_ref[r] = jnp.maximum(hb + rv_ref[r], 0.0)


def _build_hp(h, relvectors):
    return pl.pallas_call(
        _hp_body,
        grid=(N_NODES // _HP_BLK,),
        in_specs=[
            pl.BlockSpec((_HP_BLK, DIM), lambda i: (i, 0)),
            pl.BlockSpec((NUM_RELS, DIM), lambda i: (0, 0)),
        ],
        out_specs=pl.BlockSpec((NUM_RELS, _HP_BLK, DIM), lambda i: (0, i, 0)),
        out_shape=jax.ShapeDtypeStruct((NUM_RELS, N_NODES, DIM), jnp.float32),
    )(h, relvectors)


def _sc_body(hp_hbm, cmb_hbm, z_hbm,
             acc0_hbm, acc1_hbm, cnt_hbm,
             cmb_v, dst_v, gidx_v, rows_v, cnt_v, acc_sh, sem):
    c = lax.axis_index("c")
    s = lax.axis_index("s")
    wid = c * 16 + s
    row0 = s * ROWS_PER_TILE

    pltpu.sync_copy(z_hbm, acc_sh.at[pl.ds(row0, ROWS_PER_TILE)])

    zeros16 = jnp.zeros((16,), jnp.float32)
    zidx16 = jnp.zeros((16,), jnp.int32)
    ones16 = jnp.ones((16,), jnp.float32)

    @pl.loop(0, NPAD, step=16)
    def _(i):
        cnt_v[0, pl.ds(i, 16)] = zeros16

    plsc.subcore_barrier()

    @pl.loop(0, CHUNKS_PER_TILE)
    def _(ci):
        pltpu.sync_copy(cmb_hbm.at[wid, ci], cmb_v)

        @pl.loop(0, CHUNK, step=16)
        def _(j):
            sv = cmb_v[pl.ds(j, 16)]
            ev = cmb_v[pl.ds(2 * CHUNK + j, 16)]
            gidx_v[pl.ds(j, 16)] = ev * N_NODES + sv
            dv = cmb_v[pl.ds(CHUNK + j, 16)]
            dst_v[pl.ds(j, 16)] = dv
            plsc.addupdate_scatter(cnt_v, [zidx16, dv], ones16)

        pltpu.async_copy(hp_hbm.at[gidx_v], rows_v, sem).wait()
        pltpu.sync_copy(rows_v, acc_sh.at[dst_v], add=True)

    plsc.subcore_barrier()

    pltpu.sync_copy(cnt_v, cnt_hbm.at[wid])

    @pl.when(c == 0)
    def _():
        pltpu.sync_copy(acc_sh.at[pl.ds(row0, ROWS_PER_TILE)],
                        acc0_hbm.at[pl.ds(row0, ROWS_PER_TILE)])

    @pl.when(c == 1)
    def _():
        pltpu.sync_copy(acc_sh.at[pl.ds(row0, ROWS_PER_TILE)],
                        acc1_hbm.at[pl.ds(row0, ROWS_PER_TILE)])


def _sc_aggregate(hp, cmb, zrows):
    mesh = plsc.VectorSubcoreMesh(core_axis_name="c", subcore_axis_name="s")
    out_type = (
        jax.ShapeDtypeStruct((NPAD, DIM), jnp.float32),
        jax.ShapeDtypeStruct((NPAD, DIM), jnp.float32),
        jax.ShapeDtypeStruct((NUM_TILES, 1, NPAD), jnp.float32),
    )
    scratch = [
        pltpu.VMEM((3 * CHUNK,), jnp.int32),
        pltpu.VMEM((CHUNK,), jnp.int32),
        pltpu.VMEM((CHUNK,), jnp.int32),
        pltpu.VMEM((CHUNK, DIM), jnp.float32),
        pltpu.VMEM((1, NPAD), jnp.float32),
        pltpu.VMEM_SHARED((NPAD, DIM), jnp.float32),
        pltpu.SemaphoreType.DMA,
    ]
    cp = pltpu.CompilerParams()
    if "needs_layout_passes" in pltpu.CompilerParams.__dataclass_fields__:
        cp = dataclasses.replace(cp, needs_layout_passes=False)
    fn = pl.kernel(_sc_body, out_type=out_type, mesh=mesh,
                   scratch_types=scratch, compiler_params=cp)
    return fn(hp, cmb, zrows)


_FIN_BLK = 1024


def _fin_body(a0_ref, a1_ref, cnt_ref, w_ref, b_ref, out_ref):
    ssum = a0_ref[...] + a1_ref[...]
    csum = jnp.sum(cnt_ref[...], axis=0)
    eye = (lax.broadcasted_iota(jnp.int32, (DIM, DIM), 0)
           == lax.broadcasted_iota(jnp.int32, (DIM, DIM), 1)
           ).astype(jnp.float32)
    ct = lax.dot_general(eye, csum, (((1,), (1,)), ((), ())),
                         preferred_element_type=jnp.float32)
    pieces = []
    for k in range(_FIN_BLK // DIM):
        col = jnp.maximum(ct[:, k:k + 1], 1.0)
        pieces.append(ssum[k * DIM:(k + 1) * DIM, :] / col)
    red = jnp.concatenate(pieces, axis=0)
    out_ref[...] = lax.dot_general(
        red, w_ref[...], (((1,), (1,)), ((), ())),
        preferred_element_type=jnp.float32) + b_ref[...]


def _finalize(acc0, acc1, cnts, W, b2):
    return pl.pallas_call(
        _fin_body,
        grid=(NPAD // _FIN_BLK,),
        in_specs=[
            pl.BlockSpec((_FIN_BLK, DIM), lambda g: (g, 0)),
            pl.BlockSpec((_FIN_BLK, DIM), lambda g: (g, 0)),
            pl.BlockSpec((NUM_TILES, _FIN_BLK // DIM, DIM), lambda g: (0, g, 0)),
            pl.BlockSpec((DIM, DIM), lambda g: (0, 0)),
            pl.BlockSpec((1, DIM), lambda g: (0, 0)),
        ],
        out_specs=pl.BlockSpec((_FIN_BLK, DIM), lambda g: (g, 0)),
        out_shape=jax.ShapeDtypeStruct((NPAD, DIM), jnp.float32),
    )(acc0, acc1, cnts, W, b2)


@jax.jit
def kernel(h, edge_index, edge_id, W, b, relvectors):
    src = edge_index[0].astype(jnp.int32)
    dst = edge_index[1].astype(jnp.int32)
    eid = edge_id.astype(jnp.int32)
    hp = _build_hp(h, relvectors).reshape(NUM_RELS * N_NODES, DIM)
    cmb = jnp.stack([src.reshape(NUM_TILES, CHUNKS_PER_TILE, CHUNK),
                     dst.reshape(NUM_TILES, CHUNKS_PER_TILE, CHUNK),
                     eid.reshape(NUM_TILES, CHUNKS_PER_TILE, CHUNK)],
                    axis=2).reshape(NUM_TILES, CHUNKS_PER_TILE, 3 * CHUNK)
    zrows = jnp.zeros((ROWS_PER_TILE, DIM), jnp.float32)
    acc0, acc1, cnts = _sc_aggregate(hp, cmb, zrows)
    cnts = cnts.reshape(NUM_TILES, NPAD // DIM, DIM)
    out = _finalize(acc0, acc1, cnts, W, b.reshape(1, DIM))
    return out[:N_NODES]


# speedup vs baseline: 1.5660x; 1.1967x over previous
import dataclasses

import jax
import jax.numpy as jnp
from jax import lax
from jax.experimental import pallas as pl
from jax.experimental.pallas import tpu as pltpu
from jax.experimental.pallas import tpu_sc as plsc

N_NODES = 10000
N_EDGES = 320000
DIM = 128
NUM_RELS = 5

NPAD = 10240
NUM_TILES = 32
EDGES_PER_TILE = N_EDGES // NUM_TILES
CHUNK = 80
CHUNKS_PER_TILE = EDGES_PER_TILE // CHUNK
ROWS_PER_TILE = NPAD // 16


_HP_BLK = 1000


def _hp_body(h_ref, rv_ref, out_ref):
    hb = h_ref[...]
    for r in range(NUM_RELS):
        out_ref[r] = jnp.maximum(hb + rv_ref[r], 0.0)


def _build_hp(h, relvectors):
    return pl.pallas_call(
        _hp_body,
        grid=(N_NODES // _HP_BLK,),
        in_specs=[
            pl.BlockSpec((_HP_BLK, DIM), lambda i: (i, 0)),
            pl.BlockSpec((NUM_RELS, DIM), lambda i: (0, 0)),
        ],
        out_specs=pl.BlockSpec((NUM_RELS, _HP_BLK, DIM), lambda i: (0, i, 0)),
        out_shape=jax.ShapeDtypeStruct((NUM_RELS, N_NODES, DIM), jnp.float32),
    )(h, relvectors)


def _sc_body(hp_hbm, cmb_hbm, z_hbm,
             acc0_hbm, acc1_hbm, cnt_hbm,
             cmbA, cmbB, dst_v, gidx_v, rows_v, cnt_v, acc_sh,
             sem, ixA, ixB):
    cmbs = (cmbA, cmbB)
    isems = (ixA, ixB)
    c = lax.axis_index("c")
    s = lax.axis_index("s")
    wid = c * 16 + s
    row0 = s * ROWS_PER_TILE

    pltpu.sync_copy(z_hbm, acc_sh.at[pl.ds(row0, ROWS_PER_TILE)])

    zeros16 = jnp.zeros((16,), jnp.float32)
    zidx16 = jnp.zeros((16,), jnp.int32)
    ones16 = jnp.ones((16,), jnp.float32)

    @pl.loop(0, NPAD, step=16)
    def _(i):
        cnt_v[0, pl.ds(i, 16)] = zeros16

    plsc.subcore_barrier()

    pltpu.async_copy(cmb_hbm.at[wid, 0], cmbA, ixA)
    pltpu.async_copy(cmb_hbm.at[wid, 1], cmbB, ixB)

    def do_chunk(ci, b):
        cmb_v = cmbs[b]
        pltpu.make_async_copy(cmb_hbm.at[wid, 0], cmb_v, isems[b]).wait()

        @pl.loop(0, CHUNK, step=16)
        def _(j):
            sv = cmb_v[pl.ds(j, 16)]
            ev = cmb_v[pl.ds(2 * CHUNK + j, 16)]
            gidx_v[pl.ds(j, 16)] = ev * N_NODES + sv
            dv = cmb_v[pl.ds(CHUNK + j, 16)]
            dst_v[pl.ds(j, 16)] = dv
            plsc.addupdate_scatter(cnt_v, [zidx16, dv], ones16)

        @pl.when(ci + 2 < CHUNKS_PER_TILE)
        def _():
            pltpu.async_copy(cmb_hbm.at[wid, ci + 2], cmb_v, isems[b])
        pltpu.async_copy(hp_hbm.at[gidx_v], rows_v, sem).wait()
        pltpu.sync_copy(rows_v, acc_sh.at[dst_v], add=True)

    @pl.loop(0, CHUNKS_PER_TILE - 1, step=2)
    def _(g):
        do_chunk(g, 0)
        do_chunk(g + 1, 1)

    do_chunk(CHUNKS_PER_TILE - 1, 0)

    plsc.subcore_barrier()

    pltpu.sync_copy(cnt_v, cnt_hbm.at[wid])

    @pl.when(c == 0)
    def _():
        pltpu.sync_copy(acc_sh.at[pl.ds(row0, ROWS_PER_TILE)],
                        acc0_hbm.at[pl.ds(row0, ROWS_PER_TILE)])

    @pl.when(c == 1)
    def _():
        pltpu.sync_copy(acc_sh.at[pl.ds(row0, ROWS_PER_TILE)],
                        acc1_hbm.at[pl.ds(row0, ROWS_PER_TILE)])


def _sc_aggregate(hp, cmb, zrows):
    mesh = plsc.VectorSubcoreMesh(core_axis_name="c", subcore_axis_name="s")
    out_type = (
        jax.ShapeDtypeStruct((NPAD, DIM), jnp.float32),
        jax.ShapeDtypeStruct((NPAD, DIM), jnp.float32),
        jax.ShapeDtypeStruct((NUM_TILES, 1, NPAD), jnp.float32),
    )
    scratch = [
        pltpu.VMEM((3 * CHUNK,), jnp.int32),
        pltpu.VMEM((3 * CHUNK,), jnp.int32),
        pltpu.VMEM((CHUNK,), jnp.int32),
        pltpu.VMEM((CHUNK,), jnp.int32),
        pltpu.VMEM((CHUNK, DIM), jnp.float32),
        pltpu.VMEM((1, NPAD), jnp.float32),
        pltpu.VMEM_SHARED((NPAD, DIM), jnp.float32),
        pltpu.SemaphoreType.DMA,
        pltpu.SemaphoreType.DMA,
        pltpu.SemaphoreType.DMA,
    ]
    cp = pltpu.CompilerParams()
    if "needs_layout_passes" in pltpu.CompilerParams.__dataclass_fields__:
        cp = dataclasses.replace(cp, needs_layout_passes=False)
    fn = pl.kernel(_sc_body, out_type=out_type, mesh=mesh,
                   scratch_types=scratch, compiler_params=cp)
    return fn(hp, cmb, zrows)


_FIN_BLK = 1024


def _fin_body(a0_ref, a1_ref, cnt_ref, w_ref, b_ref, out_ref):
    ssum = a0_ref[...] + a1_ref[...]
    csum = jnp.sum(cnt_ref[...], axis=0)
    eye = (lax.broadcasted_iota(jnp.int32, (DIM, DIM), 0)
           == lax.broadcasted_iota(jnp.int32, (DIM, DIM), 1)
           ).astype(jnp.float32)
    ct = lax.dot_general(eye, csum, (((1,), (1,)), ((), ())),
                         preferred_element_type=jnp.float32)
    pieces = []
    for k in range(_FIN_BLK // DIM):
        col = jnp.maximum(ct[:, k:k + 1], 1.0)
        pieces.append(ssum[k * DIM:(k + 1) * DIM, :] / col)
    red = jnp.concatenate(pieces, axis=0)
    out_ref[...] = lax.dot_general(
        red, w_ref[...], (((1,), (1,)), ((), ())),
        preferred_element_type=jnp.float32) + b_ref[...]


def _finalize(acc0, acc1, cnts, W, b2):
    return pl.pallas_call(
        _fin_body,
        grid=(NPAD // _FIN_BLK,),
        in_specs=[
            pl.BlockSpec((_FIN_BLK, DIM), lambda g: (g, 0)),
            pl.BlockSpec((_FIN_BLK, DIM), lambda g: (g, 0)),
            pl.BlockSpec((NUM_TILES, _FIN_BLK // DIM, DIM), lambda g: (0, g, 0)),
            pl.BlockSpec((DIM, DIM), lambda g: (0, 0)),
            pl.BlockSpec((1, DIM), lambda g: (0, 0)),
        ],
        out_specs=pl.BlockSpec((_FIN_BLK, DIM), lambda g: (g, 0)),
        out_shape=jax.ShapeDtypeStruct((NPAD, DIM), jnp.float32),
    )(acc0, acc1, cnts, W, b2)


@jax.jit
def kernel(h, edge_index, edge_id, W, b, relvectors):
    src = edge_index[0].astype(jnp.int32)
    dst = edge_index[1].astype(jnp.int32)
    eid = edge_id.astype(jnp.int32)
    hp = _build_hp(h, relvectors).reshape(NUM_RELS * N_NODES, DIM)
    cmb = jnp.stack([src.reshape(NUM_TILES, CHUNKS_PER_TILE, CHUNK),
                     dst.reshape(NUM_TILES, CHUNKS_PER_TILE, CHUNK),
                     eid.reshape(NUM_TILES, CHUNKS_PER_TILE, CHUNK)],
                    axis=2).reshape(NUM_TILES, CHUNKS_PER_TILE, 3 * CHUNK)
    zrows = jnp.zeros((ROWS_PER_TILE, DIM), jnp.float32)
    acc0, acc1, cnts = _sc_aggregate(hp, cmb, zrows)
    cnts = cnts.reshape(NUM_TILES, NPAD // DIM, DIM)
    out = _finalize(acc0, acc1, cnts, W, b.reshape(1, DIM))
    return out[:N_NODES]


# speedup vs baseline: 2.2592x; 1.4427x over previous
import dataclasses

import jax
import jax.numpy as jnp
from jax import lax
from jax.experimental import pallas as pl
from jax.experimental.pallas import tpu as pltpu
from jax.experimental.pallas import tpu_sc as plsc

N_NODES = 10000
N_EDGES = 320000
DIM = 128
NUM_RELS = 5

NPAD = 10240
NUM_TILES = 32
EDGES_PER_TILE = N_EDGES // NUM_TILES
CHUNK = 80
CHUNKS_PER_TILE = EDGES_PER_TILE // CHUNK
ROWS_PER_TILE = NPAD // 16


_HP_BLK = 1000


def _hp_body(h_ref, rv_ref, out_ref):
    hb = h_ref[...]
    for r in range(NUM_RELS):
        out_ref[r] = jnp.maximum(hb + rv_ref[r], 0.0)


def _build_hp(h, relvectors):
    return pl.pallas_call(
        _hp_body,
        grid=(N_NODES // _HP_BLK,),
        in_specs=[
            pl.BlockSpec((_HP_BLK, DIM), lambda i: (i, 0)),
            pl.BlockSpec((NUM_RELS, DIM), lambda i: (0, 0)),
        ],
        out_specs=pl.BlockSpec((NUM_RELS, _HP_BLK, DIM), lambda i: (0, i, 0)),
        out_shape=jax.ShapeDtypeStruct((NUM_RELS, N_NODES, DIM), jnp.float32),
    )(h, relvectors)


def _sc_body(hp_hbm, cmb_hbm, z_hbm,
             acc0_hbm, acc1_hbm, cnt_hbm,
             cmbA, cmbB, dsA, dsB, gxA, gxB, rowsA, rowsB, cnt_v, acc_sh,
             gsA, gsB, ixA, ixB):
    cmbs = (cmbA, cmbB)
    dsts = (dsA, dsB)
    gidxs = (gxA, gxB)
    rows = (rowsA, rowsB)
    gsems = (gsA, gsB)
    isems = (ixA, ixB)
    c = lax.axis_index("c")
    s = lax.axis_index("s")
    wid = c * 16 + s
    row0 = s * ROWS_PER_TILE

    pltpu.sync_copy(z_hbm, acc_sh.at[pl.ds(row0, ROWS_PER_TILE)])

    zeros16 = jnp.zeros((16,), jnp.float32)
    zidx16 = jnp.zeros((16,), jnp.int32)
    ones16 = jnp.ones((16,), jnp.float32)

    @pl.loop(0, NPAD, step=16)
    def _(i):
        cnt_v[0, pl.ds(i, 16)] = zeros16

    plsc.subcore_barrier()

    pltpu.async_copy(cmb_hbm.at[wid, 0], cmbA, ixA)
    pltpu.async_copy(cmb_hbm.at[wid, 1], cmbB, ixB)

    def prep(ci, b):
        cmb_v = cmbs[b]
        gidx_v = gidxs[b]
        dst_v = dsts[b]
        pltpu.make_async_copy(cmb_hbm.at[wid, 0], cmb_v, isems[b]).wait()

        @pl.loop(0, CHUNK, step=16)
        def _(j):
            sv = cmb_v[pl.ds(j, 16)]
            ev = cmb_v[pl.ds(2 * CHUNK + j, 16)]
            gidx_v[pl.ds(j, 16)] = ev * N_NODES + sv
            dv = cmb_v[pl.ds(CHUNK + j, 16)]
            dst_v[pl.ds(j, 16)] = dv
            plsc.addupdate_scatter(cnt_v, [zidx16, dv], ones16)

        @pl.when(ci + 2 < CHUNKS_PER_TILE)
        def _():
            pltpu.async_copy(cmb_hbm.at[wid, ci + 2], cmb_v, isems[b])
        pltpu.async_copy(hp_hbm.at[gidx_v], rows[b], gsems[b])

    def drain(b):
        pltpu.make_async_copy(hp_hbm.at[pl.ds(0, CHUNK)], rows[b],
                              gsems[b]).wait()
        pltpu.sync_copy(rows[b], acc_sh.at[dsts[b]], add=True)

    prep(0, 0)

    @pl.loop(0, CHUNKS_PER_TILE - 1, step=2)
    def _(g):
        prep(g + 1, 1)
        drain(0)
        prep(g + 2, 0)
        drain(1)

    drain(0)

    plsc.subcore_barrier()

    pltpu.sync_copy(cnt_v, cnt_hbm.at[wid])

    @pl.when(c == 0)
    def _():
        pltpu.sync_copy(acc_sh.at[pl.ds(row0, ROWS_PER_TILE)],
                        acc0_hbm.at[pl.ds(row0, ROWS_PER_TILE)])

    @pl.when(c == 1)
    def _():
        pltpu.sync_copy(acc_sh.at[pl.ds(row0, ROWS_PER_TILE)],
                        acc1_hbm.at[pl.ds(row0, ROWS_PER_TILE)])


def _sc_aggregate(hp, cmb, zrows):
    mesh = plsc.VectorSubcoreMesh(core_axis_name="c", subcore_axis_name="s")
    out_type = (
        jax.ShapeDtypeStruct((NPAD, DIM), jnp.float32),
        jax.ShapeDtypeStruct((NPAD, DIM), jnp.float32),
        jax.ShapeDtypeStruct((NUM_TILES, 1, NPAD), jnp.float32),
    )
    scratch = [
        pltpu.VMEM((3 * CHUNK,), jnp.int32),
        pltpu.VMEM((3 * CHUNK,), jnp.int32),
        pltpu.VMEM((CHUNK,), jnp.int32),
        pltpu.VMEM((CHUNK,), jnp.int32),
        pltpu.VMEM((CHUNK,), jnp.int32),
        pltpu.VMEM((CHUNK,), jnp.int32),
        pltpu.VMEM((CHUNK, DIM), jnp.float32),
        pltpu.VMEM((CHUNK, DIM), jnp.float32),
        pltpu.VMEM((1, NPAD), jnp.float32),
        pltpu.VMEM_SHARED((NPAD, DIM), jnp.float32),
        pltpu.SemaphoreType.DMA,
        pltpu.SemaphoreType.DMA,
        pltpu.SemaphoreType.DMA,
        pltpu.SemaphoreType.DMA,
    ]
    cp = pltpu.CompilerParams()
    if "needs_layout_passes" in pltpu.CompilerParams.__dataclass_fields__:
        cp = dataclasses.replace(cp, needs_layout_passes=False)
    fn = pl.kernel(_sc_body, out_type=out_type, mesh=mesh,
                   scratch_types=scratch, compiler_params=cp)
    return fn(hp, cmb, zrows)


_FIN_BLK = 1024


def _fin_body(a0_ref, a1_ref, cnt_ref, w_ref, b_ref, out_ref):
    ssum = a0_ref[...] + a1_ref[...]
    csum = jnp.sum(cnt_ref[...], axis=0)
    eye = (lax.broadcasted_iota(jnp.int32, (DIM, DIM), 0)
           == lax.broadcasted_iota(jnp.int32, (DIM, DIM), 1)
           ).astype(jnp.float32)
    ct = lax.dot_general(eye, csum, (((1,), (1,)), ((), ())),
                         preferred_element_type=jnp.float32)
    pieces = []
    for k in range(_FIN_BLK // DIM):
        col = jnp.maximum(ct[:, k:k + 1], 1.0)
        pieces.append(ssum[k * DIM:(k + 1) * DIM, :] / col)
    red = jnp.concatenate(pieces, axis=0)
    out_ref[...] = lax.dot_general(
        red, w_ref[...], (((1,), (1,)), ((), ())),
        preferred_element_type=jnp.float32) + b_ref[...]


def _finalize(acc0, acc1, cnts, W, b2):
    return pl.pallas_call(
        _fin_body,
        grid=(NPAD // _FIN_BLK,),
        in_specs=[
            pl.BlockSpec((_FIN_BLK, DIM), lambda g: (g, 0)),
            pl.BlockSpec((_FIN_BLK, DIM), lambda g: (g, 0)),
            pl.BlockSpec((NUM_TILES, _FIN_BLK // DIM, DIM), lambda g: (0, g, 0)),
            pl.BlockSpec((DIM, DIM), lambda g: (0, 0)),
            pl.BlockSpec((1, DIM), lambda g: (0, 0)),
        ],
        out_specs=pl.BlockSpec((_FIN_BLK, DIM), lambda g: (g, 0)),
        out_shape=jax.ShapeDtypeStruct((NPAD, DIM), jnp.float32),
    )(acc0, acc1, cnts, W, b2)


@jax.jit
def kernel(h, edge_index, edge_id, W, b, relvectors):
    src = edge_index[0].astype(jnp.int32)
    dst = edge_index[1].astype(jnp.int32)
    eid = edge_id.astype(jnp.int32)
    hp = _build_hp(h, relvectors).reshape(NUM_RELS * N_NODES, DIM)
    cmb = jnp.stack([src.reshape(NUM_TILES, CHUNKS_PER_TILE, CHUNK),
                     dst.reshape(NUM_TILES, CHUNKS_PER_TILE, CHUNK),
                     eid.reshape(NUM_TILES, CHUNKS_PER_TILE, CHUNK)],
                    axis=2).reshape(NUM_TILES, CHUNKS_PER_TILE, 3 * CHUNK)
    zrows = jnp.zeros((ROWS_PER_TILE, DIM), jnp.float32)
    acc0, acc1, cnts = _sc_aggregate(hp, cmb, zrows)
    cnts = cnts.reshape(NUM_TILES, NPAD // DIM, DIM)
    out = _finalize(acc0, acc1, cnts, W, b.reshape(1, DIM))
    return out[:N_NODES]
